# baseline (speedup 1.0000x reference)
import functools

import jax
import jax.numpy as jnp
from jax import lax
from jax.experimental import pallas as pl
from jax.experimental.pallas import tpu as pltpu
from jax.experimental.pallas import tpu_sc as plsc

N_GATES = 96
N_SCALARS = 128
N_TENSOR = 352
IN_DIM = 576
OUT_DIM = 480
BATCH = 100000

NUM_CORES = 2
NUM_SUBCORES = 16
NW = NUM_CORES * NUM_SUBCORES
ROWS_PER_W = BATCH // NW
CHUNK = 25
NCHUNK = ROWS_PER_W // CHUNK

_TENS0 = N_GATES + N_SCALARS

_GD = lax.GatherDimensionNumbers(
    offset_dims=(), collapsed_slice_dims=(0,), start_index_map=(0,))


def _vgather(s, idx):
    return lax.gather(s, idx[:, None], _GD, slice_sizes=(1,),
                      mode=lax.GatherScatterMode.PROMISE_IN_BOUNDS)


def _sc_body(x_hbm, out_hbm, in_v, out_v):
    wid = lax.axis_index("s") * NUM_CORES + lax.axis_index("c")

    def chunk_body(c, carry):
        row0 = wid * ROWS_PER_W + c * CHUNK
        pltpu.sync_copy(x_hbm.at[pl.ds(row0 * IN_DIM, CHUNK * IN_DIM)], in_v)

        def row_body(r, carry2):
            ib = r * IN_DIM
            ob = r * OUT_DIM
            iota = lax.iota(jnp.int32, 16)
            three = jnp.full((16,), 3, jnp.int32)
            five = jnp.full((16,), 5, jnp.int32)
            idx3 = [lax.div(iota + 16 * m, three) for m in range(3)]
            idx5 = [lax.div(iota + 16 * m, five) for m in range(5)]
            for k in range(N_SCALARS // 16):
                v = in_v[pl.ds(ib + N_GATES + 16 * k, 16)]
                out_v[pl.ds(ob + 16 * k, 16)] = v / (jnp.exp(-v) + 1.0)
            for k in range(4):
                g = in_v[pl.ds(ib + 16 * k, 16)]
                s = 1.0 / (jnp.exp(-g) + 1.0)
                for m in range(3):
                    j = 48 * k + 16 * m
                    ge = _vgather(s, idx3[m])
                    t = in_v[pl.ds(ib + _TENS0 + j, 16)]
                    out_v[pl.ds(ob + N_SCALARS + j, 16)] = ge * t
            for k in range(2):
                g = in_v[pl.ds(ib + 64 + 16 * k, 16)]
                s = 1.0 / (jnp.exp(-g) + 1.0)
                for m in range(5):
                    j = 192 + 80 * k + 16 * m
                    ge = _vgather(s, idx5[m])
                    t = in_v[pl.ds(ib + _TENS0 + j, 16)]
                    out_v[pl.ds(ob + N_SCALARS + j, 16)] = ge * t
            return carry2

        lax.fori_loop(0, CHUNK, row_body, 0)
        pltpu.sync_copy(out_v, out_hbm.at[pl.ds(row0 * OUT_DIM, CHUNK * OUT_DIM)])
        return carry

    lax.fori_loop(0, NCHUNK, chunk_body, 0)


_sc_gate = functools.partial(
    pl.kernel,
    mesh=plsc.VectorSubcoreMesh(core_axis_name="c", subcore_axis_name="s"),
    out_type=jax.ShapeDtypeStruct((BATCH * OUT_DIM,), jnp.float32),
    scratch_types=[
        pltpu.VMEM((CHUNK * IN_DIM,), jnp.float32),
        pltpu.VMEM((CHUNK * OUT_DIM,), jnp.float32),
    ],
)(_sc_body)


def kernel(x):
    out = _sc_gate(x.reshape(-1))
    return out.reshape(BATCH, OUT_DIM)


# speedup vs baseline: 1.1107x; 1.1107x over previous
import functools

import jax
import jax.numpy as jnp
from jax import lax
from jax.experimental import pallas as pl
from jax.experimental.pallas import tpu as pltpu
from jax.experimental.pallas import tpu_sc as plsc

N_GATES = 96
N_SCALARS = 128
N_TENSOR = 352
IN_DIM = 576
OUT_DIM = 480
BATCH = 100000

NUM_CORES = 2
NUM_SUBCORES = 16
NW = NUM_CORES * NUM_SUBCORES
ROWS_PER_W = BATCH // NW
CHUNK = 25
NCHUNK = ROWS_PER_W // CHUNK

_TENS0 = N_GATES + N_SCALARS

_GD = lax.GatherDimensionNumbers(
    offset_dims=(), collapsed_slice_dims=(0,), start_index_map=(0,))


def _vgather(s, idx):
    return lax.gather(s, idx[:, None], _GD, slice_sizes=(1,),
                      mode=lax.GatherScatterMode.PROMISE_IN_BOUNDS)


def _sc_body(x_hbm, out_hbm, in_v, out_v):
    wid = lax.axis_index("s") * NUM_CORES + lax.axis_index("c")

    def chunk_body(c, carry):
        row0 = wid * ROWS_PER_W + c * CHUNK
        pltpu.sync_copy(x_hbm.at[pl.ds(row0 * IN_DIM, CHUNK * IN_DIM)], in_v)

        iota = lax.iota(jnp.int32, 16)
        three = jnp.full((16,), 3, jnp.int32)
        five = jnp.full((16,), 5, jnp.int32)
        idx3 = [lax.div(iota + 16 * m, three) for m in range(3)]
        idx5 = [lax.div(iota + 16 * m, five) for m in range(5)]

        for r in range(CHUNK):
            ib = r * IN_DIM
            ob = r * OUT_DIM
            for k in range(N_SCALARS // 16):
                v = in_v[pl.ds(ib + N_GATES + 16 * k, 16)]
                out_v[pl.ds(ob + 16 * k, 16)] = v / (jnp.exp(-v) + 1.0)
            for k in range(4):
                g = in_v[pl.ds(ib + 16 * k, 16)]
                s = 1.0 / (jnp.exp(-g) + 1.0)
                for m in range(3):
                    j = 48 * k + 16 * m
                    ge = _vgather(s, idx3[m])
                    t = in_v[pl.ds(ib + _TENS0 + j, 16)]
                    out_v[pl.ds(ob + N_SCALARS + j, 16)] = ge * t
            for k in range(2):
                g = in_v[pl.ds(ib + 64 + 16 * k, 16)]
                s = 1.0 / (jnp.exp(-g) + 1.0)
                for m in range(5):
                    j = 192 + 80 * k + 16 * m
                    ge = _vgather(s, idx5[m])
                    t = in_v[pl.ds(ib + _TENS0 + j, 16)]
                    out_v[pl.ds(ob + N_SCALARS + j, 16)] = ge * t

        pltpu.sync_copy(out_v, out_hbm.at[pl.ds(row0 * OUT_DIM, CHUNK * OUT_DIM)])
        return carry

    lax.fori_loop(0, NCHUNK, chunk_body, 0)


_sc_gate = functools.partial(
    pl.kernel,
    mesh=plsc.VectorSubcoreMesh(core_axis_name="c", subcore_axis_name="s"),
    out_type=jax.ShapeDtypeStruct((BATCH * OUT_DIM,), jnp.float32),
    scratch_types=[
        pltpu.VMEM((CHUNK * IN_DIM,), jnp.float32),
        pltpu.VMEM((CHUNK * OUT_DIM,), jnp.float32),
    ],
)(_sc_body)


def kernel(x):
    out = _sc_gate(x.reshape(-1))
    return out.reshape(BATCH, OUT_DIM)


# speedup vs baseline: 2.6903x; 2.4222x over previous
import functools

import jax
import jax.numpy as jnp
from jax import lax
from jax.experimental import pallas as pl
from jax.experimental.pallas import tpu as pltpu
from jax.experimental.pallas import tpu_sc as plsc

N_GATES = 96
N_SCALARS = 128
N_TENSOR = 352
IN_DIM = 576
OUT_DIM = 480
BATCH = 100000

NUM_CORES = 2
NUM_SUBCORES = 16
NW = NUM_CORES * NUM_SUBCORES
CHUNK = 32
NCHUNKS = BATCH // CHUNK

_TENS0 = N_GATES + N_SCALARS

_GD = lax.GatherDimensionNumbers(
    offset_dims=(), collapsed_slice_dims=(0,), start_index_map=(0,))


def _vgather(s, idx):
    return lax.gather(s, idx[:, None], _GD, slice_sizes=(1,),
                      mode=lax.GatherScatterMode.PROMISE_IN_BOUNDS)


def _sc_body(x_hbm, out_hbm, in_v, out_v):
    wid = lax.axis_index("s") * NUM_CORES + lax.axis_index("c")
    ntrips = lax.div(jnp.int32(NCHUNKS + NW - 1) - wid, jnp.int32(NW))

    def chunk_body(i, carry):
        row0 = (wid + i * NW) * CHUNK
        pltpu.sync_copy(x_hbm.at[pl.ds(row0, CHUNK)], in_v)

        iota = lax.iota(jnp.int32, 16)
        three = jnp.full((16,), 3, jnp.int32)
        five = jnp.full((16,), 5, jnp.int32)
        idx3 = [lax.div(iota + 16 * m, three) for m in range(3)]
        idx5 = [lax.div(iota + 16 * m, five) for m in range(5)]

        for r in range(CHUNK):
            for k in range(N_SCALARS // 16):
                v = in_v[r, pl.ds(N_GATES + 16 * k, 16)]
                out_v[r, pl.ds(16 * k, 16)] = v / (jnp.exp(-v) + 1.0)
            for k in range(4):
                g = in_v[r, pl.ds(16 * k, 16)]
                s = 1.0 / (jnp.exp(-g) + 1.0)
                for m in range(3):
                    j = 48 * k + 16 * m
                    ge = _vgather(s, idx3[m])
                    t = in_v[r, pl.ds(_TENS0 + j, 16)]
                    out_v[r, pl.ds(N_SCALARS + j, 16)] = ge * t
            for k in range(2):
                g = in_v[r, pl.ds(64 + 16 * k, 16)]
                s = 1.0 / (jnp.exp(-g) + 1.0)
                for m in range(5):
                    j = 192 + 80 * k + 16 * m
                    ge = _vgather(s, idx5[m])
                    t = in_v[r, pl.ds(_TENS0 + j, 16)]
                    out_v[r, pl.ds(N_SCALARS + j, 16)] = ge * t

        pltpu.sync_copy(out_v, out_hbm.at[pl.ds(row0, CHUNK)])
        return carry

    lax.fori_loop(0, ntrips, chunk_body, 0)


_sc_gate = functools.partial(
    pl.kernel,
    mesh=plsc.VectorSubcoreMesh(core_axis_name="c", subcore_axis_name="s"),
    out_type=jax.ShapeDtypeStruct((BATCH, OUT_DIM), jnp.float32),
    scratch_types=[
        pltpu.VMEM((CHUNK, IN_DIM), jnp.float32),
        pltpu.VMEM((CHUNK, OUT_DIM), jnp.float32),
    ],
)(_sc_body)


def kernel(x):
    return _sc_gate(x)
